# baseline (speedup 1.0000x reference)
import functools

import jax
import jax.numpy as jnp
from jax import lax
from jax.experimental import pallas as pl
from jax.experimental.pallas import tpu as pltpu
from jax.experimental.pallas import tpu_sc as plsc

N_NODES = 10000
FDIM = 128
CHUNK = 128
NC = 2
NS = 16
NW = NC * NS
ROWS_MAIN = (N_NODES // NS) // 8 * 8
ROWS_TAIL = N_NODES - NS * ROWS_MAIN
GRP = 8
SUP = 1


def _sc_aggregate(row2d, col2d, val2d, features):
    cpw = row2d.shape[0] // NW

    mesh = plsc.VectorSubcoreMesh(core_axis_name="c", subcore_axis_name="s")

    @functools.partial(
        pl.kernel,
        mesh=mesh,
        out_type=jax.ShapeDtypeStruct((NC, N_NODES, FDIM), jnp.float32),
        scratch_types=[
            pltpu.VMEM_SHARED((N_NODES, FDIM), jnp.float32),
            pltpu.VMEM((GRP, CHUNK), jnp.int32),
            pltpu.VMEM((GRP, CHUNK), jnp.int32),
            pltpu.VMEM((GRP, CHUNK), jnp.float32),
            pltpu.VMEM((SUP, CHUNK, FDIM), jnp.float32),
            pltpu.SemaphoreType.DMA,
        ],
    )
    def agg(row_hbm, col_hbm, val_hbm, feat_hbm, out_hbm, acc, colv, rowv,
            valv, grows, gsem):
        cc = lax.axis_index("c")
        sid = lax.axis_index("s")
        wid = sid * NC + cc
        chunk_base = wid * cpw

        def zrow(r, _):
            def zcol(i, _):
                grows[0, r, pl.ds(i * 16, 16)] = jnp.zeros((16,), jnp.float32)
                return 0
            return lax.fori_loop(0, FDIM // 16, zcol, 0)
        lax.fori_loop(0, CHUNK, zrow, 0)

        base_row = sid * ROWS_MAIN
        for j in range(ROWS_MAIN // CHUNK):
            pltpu.sync_copy(grows.at[0],
                            acc.at[pl.ds(base_row + j * CHUNK, CHUNK)])
        tail = ROWS_MAIN - (ROWS_MAIN // CHUNK) * CHUNK
        pltpu.sync_copy(
            grows.at[0, pl.ds(0, tail)],
            acc.at[pl.ds(base_row + (ROWS_MAIN // CHUNK) * CHUNK, tail)])

        @pl.when(sid == 0)
        def _():
            pltpu.sync_copy(grows.at[0, pl.ds(0, ROWS_TAIL)],
                            acc.at[pl.ds(NS * ROWS_MAIN, ROWS_TAIL)])
        plsc.subcore_barrier()

        def group_body(g, _):
            gbase = chunk_base + g * GRP
            pltpu.sync_copy(col_hbm.at[pl.ds(gbase, GRP)], colv)
            pltpu.sync_copy(row_hbm.at[pl.ds(gbase, GRP)], rowv)
            pltpu.sync_copy(val_hbm.at[pl.ds(gbase, GRP)], valv)
            for s in range(GRP // SUP):
                pltpu.async_copy(feat_hbm.at[colv.at[s]], grows.at[0],
                                 gsem).wait()

                def grp_body(grp, _):
                    vrow = s * SUP + grp // 8
                    vv = valv[vrow, pl.ds((grp % 8) * 16, 16)]
                    sb = grp // 8
                    for lane in range(16):
                        v = vv[lane]
                        e = (grp % 8) * 16 + lane
                        for kk in range(FDIM // 16):
                            grows[sb, e, pl.ds(kk * 16, 16)] = (
                                grows[sb, e, pl.ds(kk * 16, 16)] * v)
                    return 0
                lax.fori_loop(0, SUP * CHUNK // 16, grp_body, 0)

                pltpu.sync_copy(grows.at[0], acc.at[rowv.at[s]], add=True)
            return 0

        lax.fori_loop(0, cpw // GRP, group_body, 0)
        plsc.subcore_barrier()

        pltpu.sync_copy(acc.at[pl.ds(base_row, ROWS_MAIN)],
                        out_hbm.at[cc, pl.ds(base_row, ROWS_MAIN)])

        @pl.when(sid == 0)
        def _():
            pltpu.sync_copy(acc.at[pl.ds(NS * ROWS_MAIN, ROWS_TAIL)],
                            out_hbm.at[cc, pl.ds(NS * ROWS_MAIN, ROWS_TAIL)])

    return agg(row2d, col2d, val2d, features)


def _tc_combine_matmul(partials, weight):
    bn = 1000

    def body(p_ref, w_ref, o_ref):
        s = p_ref[0] + p_ref[1]
        o_ref[...] = jnp.maximum(
            jnp.dot(s, w_ref[...], preferred_element_type=jnp.float32), 0.0)

    return pl.pallas_call(
        body,
        grid=(N_NODES // bn,),
        in_specs=[
            pl.BlockSpec((NC, bn, FDIM), lambda i: (0, i, 0)),
            pl.BlockSpec((FDIM, FDIM), lambda i: (0, 0)),
        ],
        out_specs=pl.BlockSpec((bn, FDIM), lambda i: (i, 0)),
        out_shape=jax.ShapeDtypeStruct((N_NODES, FDIM), jnp.float32),
    )(partials, weight)


def kernel(features, adj_indices, adj_values, weight):
    idx = adj_indices.astype(jnp.int32)
    n_edges = idx.shape[1]
    gran = NW * GRP * CHUNK
    n_pad = (n_edges + gran - 1) // gran * gran
    pad = n_pad - n_edges
    n_chunks = n_pad // CHUNK
    row = jnp.pad(idx[0], (0, pad)).reshape(n_chunks, CHUNK)
    col = jnp.pad(idx[1], (0, pad)).reshape(n_chunks, CHUNK)
    val = jnp.pad(adj_values, (0, pad)).reshape(n_chunks, CHUNK)
    partials = _sc_aggregate(row, col, val, features)
    return _tc_combine_matmul(partials, weight)


# speedup vs baseline: 1.0068x; 1.0068x over previous
import functools

import jax
import jax.numpy as jnp
from jax import lax
from jax.experimental import pallas as pl
from jax.experimental.pallas import tpu as pltpu
from jax.experimental.pallas import tpu_sc as plsc

N_NODES = 10000
FDIM = 128
CHUNK = 128
NC = 2
NS = 16
NW = NC * NS
ROWS_MAIN = (N_NODES // NS) // 8 * 8
ROWS_TAIL = N_NODES - NS * ROWS_MAIN
GRP = 8
SUP = 1


def _sc_aggregate(row2d, col2d, val2d, features):
    cpw = row2d.shape[0] // NW

    mesh = plsc.VectorSubcoreMesh(core_axis_name="c", subcore_axis_name="s")

    @functools.partial(
        pl.kernel,
        mesh=mesh,
        out_type=jax.ShapeDtypeStruct((NC, N_NODES, FDIM), jnp.float32),
        scratch_types=[
            pltpu.VMEM_SHARED((N_NODES, FDIM), jnp.float32),
            pltpu.VMEM((GRP, CHUNK), jnp.int32),
            pltpu.VMEM((GRP, CHUNK), jnp.int32),
            pltpu.VMEM((GRP, CHUNK), jnp.float32),
            pltpu.VMEM((SUP, CHUNK, FDIM), jnp.float32),
            pltpu.SemaphoreType.DMA,
        ],
    )
    def agg(row_hbm, col_hbm, val_hbm, feat_hbm, out_hbm, acc, colv, rowv,
            valv, grows, gsem):
        cc = lax.axis_index("c")
        sid = lax.axis_index("s")
        wid = sid * NC + cc
        chunk_base = wid * cpw

        def zrow(r, _):
            def zcol(i, _):
                grows[0, r, pl.ds(i * 16, 16)] = jnp.zeros((16,), jnp.float32)
                return 0
            return lax.fori_loop(0, FDIM // 16, zcol, 0)
        lax.fori_loop(0, CHUNK, zrow, 0)

        base_row = sid * ROWS_MAIN
        for j in range(ROWS_MAIN // CHUNK):
            pltpu.sync_copy(grows.at[0],
                            acc.at[pl.ds(base_row + j * CHUNK, CHUNK)])
        tail = ROWS_MAIN - (ROWS_MAIN // CHUNK) * CHUNK
        pltpu.sync_copy(
            grows.at[0, pl.ds(0, tail)],
            acc.at[pl.ds(base_row + (ROWS_MAIN // CHUNK) * CHUNK, tail)])

        @pl.when(sid == 0)
        def _():
            pltpu.sync_copy(grows.at[0, pl.ds(0, ROWS_TAIL)],
                            acc.at[pl.ds(NS * ROWS_MAIN, ROWS_TAIL)])
        plsc.subcore_barrier()

        def group_body(g, _):
            gbase = chunk_base + g * GRP
            pltpu.sync_copy(col_hbm.at[pl.ds(gbase, GRP)], colv)
            pltpu.sync_copy(row_hbm.at[pl.ds(gbase, GRP)], rowv)
            pltpu.sync_copy(val_hbm.at[pl.ds(gbase, GRP)], valv)
            def chunk_body(s, _):
                pltpu.async_copy(feat_hbm.at[colv.at[s]], grows.at[0],
                                 gsem).wait()

                def grp_body(grp, _):
                    vv = valv[s, pl.ds(grp * 16, 16)]
                    for lane in range(16):
                        v = vv[lane]
                        e = grp * 16 + lane
                        for kk in range(FDIM // 16):
                            grows[0, e, pl.ds(kk * 16, 16)] = (
                                grows[0, e, pl.ds(kk * 16, 16)] * v)
                    return 0
                lax.fori_loop(0, CHUNK // 16, grp_body, 0)

                pltpu.sync_copy(grows.at[0], acc.at[rowv.at[s]], add=True)
                return 0

            lax.fori_loop(0, GRP, chunk_body, 0)
            return 0

        lax.fori_loop(0, cpw // GRP, group_body, 0)
        plsc.subcore_barrier()

        pltpu.sync_copy(acc.at[pl.ds(base_row, ROWS_MAIN)],
                        out_hbm.at[cc, pl.ds(base_row, ROWS_MAIN)])

        @pl.when(sid == 0)
        def _():
            pltpu.sync_copy(acc.at[pl.ds(NS * ROWS_MAIN, ROWS_TAIL)],
                            out_hbm.at[cc, pl.ds(NS * ROWS_MAIN, ROWS_TAIL)])

    return agg(row2d, col2d, val2d, features)


def _tc_combine_matmul(partials, weight):
    bn = 1000

    def body(p_ref, w_ref, o_ref):
        s = p_ref[0] + p_ref[1]
        o_ref[...] = jnp.maximum(
            jnp.dot(s, w_ref[...], preferred_element_type=jnp.float32), 0.0)

    return pl.pallas_call(
        body,
        grid=(N_NODES // bn,),
        in_specs=[
            pl.BlockSpec((NC, bn, FDIM), lambda i: (0, i, 0)),
            pl.BlockSpec((FDIM, FDIM), lambda i: (0, 0)),
        ],
        out_specs=pl.BlockSpec((bn, FDIM), lambda i: (i, 0)),
        out_shape=jax.ShapeDtypeStruct((N_NODES, FDIM), jnp.float32),
    )(partials, weight)


def kernel(features, adj_indices, adj_values, weight):
    idx = adj_indices.astype(jnp.int32)
    n_edges = idx.shape[1]
    gran = NW * GRP * CHUNK
    n_pad = (n_edges + gran - 1) // gran * gran
    pad = n_pad - n_edges
    n_chunks = n_pad // CHUNK
    row = jnp.pad(idx[0], (0, pad)).reshape(n_chunks, CHUNK)
    col = jnp.pad(idx[1], (0, pad)).reshape(n_chunks, CHUNK)
    val = jnp.pad(adj_values, (0, pad)).reshape(n_chunks, CHUNK)
    partials = _sc_aggregate(row, col, val, features)
    return _tc_combine_matmul(partials, weight)


# speedup vs baseline: 2.4875x; 2.4706x over previous
import functools

import jax
import jax.numpy as jnp
from jax import lax
from jax.experimental import pallas as pl
from jax.experimental.pallas import tpu as pltpu
from jax.experimental.pallas import tpu_sc as plsc

N_NODES = 10000
FDIM = 128
CHUNK = 128
NC = 2
NS = 16
NW = NC * NS
ROWS_MAIN = (N_NODES // NS) // 8 * 8
ROWS_TAIL = N_NODES - NS * ROWS_MAIN
GRP = 8
SUP = 1


def _sc_aggregate(row2d, col2d, val2d, features):
    cpw = row2d.shape[0] // NW

    mesh = plsc.VectorSubcoreMesh(core_axis_name="c", subcore_axis_name="s")

    @functools.partial(
        pl.kernel,
        mesh=mesh,
        out_type=jax.ShapeDtypeStruct((NC, N_NODES, FDIM), jnp.float32),
        scratch_types=[
            pltpu.VMEM_SHARED((N_NODES, FDIM), jnp.float32),
            pltpu.VMEM((GRP, CHUNK), jnp.int32),
            pltpu.VMEM((GRP, CHUNK), jnp.int32),
            pltpu.VMEM((GRP, CHUNK), jnp.float32),
            pltpu.VMEM((SUP, CHUNK, FDIM), jnp.float32),
            pltpu.SemaphoreType.DMA,
        ],
    )
    def agg(row_hbm, col_hbm, val_hbm, feat_hbm, out_hbm, acc, colv, rowv,
            valv, grows, gsem):
        cc = lax.axis_index("c")
        sid = lax.axis_index("s")
        wid = sid * NC + cc
        chunk_base = wid * cpw

        def zrow(r, _):
            def zcol(i, _):
                grows[0, r, pl.ds(i * 16, 16)] = jnp.zeros((16,), jnp.float32)
                return 0
            return lax.fori_loop(0, FDIM // 16, zcol, 0)
        lax.fori_loop(0, CHUNK, zrow, 0)

        base_row = sid * ROWS_MAIN
        for j in range(ROWS_MAIN // CHUNK):
            pltpu.sync_copy(grows.at[0],
                            acc.at[pl.ds(base_row + j * CHUNK, CHUNK)])
        tail = ROWS_MAIN - (ROWS_MAIN // CHUNK) * CHUNK
        pltpu.sync_copy(
            grows.at[0, pl.ds(0, tail)],
            acc.at[pl.ds(base_row + (ROWS_MAIN // CHUNK) * CHUNK, tail)])

        @pl.when(sid == 0)
        def _():
            pltpu.sync_copy(grows.at[0, pl.ds(0, ROWS_TAIL)],
                            acc.at[pl.ds(NS * ROWS_MAIN, ROWS_TAIL)])
        plsc.subcore_barrier()

        def group_body(g, _):
            gbase = chunk_base + g * GRP
            pltpu.sync_copy(col_hbm.at[pl.ds(gbase, GRP)], colv)
            pltpu.sync_copy(row_hbm.at[pl.ds(gbase, GRP)], rowv)
            pltpu.sync_copy(val_hbm.at[pl.ds(gbase, GRP)], valv)
            def chunk_body(s, _):
                pltpu.async_copy(feat_hbm.at[colv.at[s]], grows.at[0],
                                 gsem).wait()

                def grp_body(grp, _):
                    vv = valv[s, pl.ds(grp * 16, 16)]
                    for lane in range(16):
                        v = vv[lane]
                        e = grp * 16 + lane
                        for kk in range(FDIM // 16):
                            grows[0, e, pl.ds(kk * 16, 16)] = (
                                grows[0, e, pl.ds(kk * 16, 16)] * v)
                    return 0
                lax.fori_loop(0, CHUNK // 16, grp_body, 0)

                pltpu.sync_copy(grows.at[0], acc.at[rowv.at[s]], add=True)
                return 0

            lax.fori_loop(0, GRP, chunk_body, 0)
            return 0

        lax.fori_loop(0, cpw // GRP, group_body, 0)
        plsc.subcore_barrier()

        pltpu.sync_copy(acc.at[pl.ds(base_row, ROWS_MAIN)],
                        out_hbm.at[cc, pl.ds(base_row, ROWS_MAIN)])

        @pl.when(sid == 0)
        def _():
            pltpu.sync_copy(acc.at[pl.ds(NS * ROWS_MAIN, ROWS_TAIL)],
                            out_hbm.at[cc, pl.ds(NS * ROWS_MAIN, ROWS_TAIL)])

    return agg(row2d, col2d, val2d, features)


def _tc_combine_matmul(partials, weight):
    bn = 1000

    def body(p_ref, w_ref, o_ref):
        s = p_ref[0] + p_ref[1]
        o_ref[...] = jnp.maximum(
            jnp.dot(s, w_ref[...], preferred_element_type=jnp.float32), 0.0)

    return pl.pallas_call(
        body,
        grid=(N_NODES // bn,),
        in_specs=[
            pl.BlockSpec((NC, bn, FDIM), lambda i: (0, i, 0)),
            pl.BlockSpec((FDIM, FDIM), lambda i: (0, 0)),
        ],
        out_specs=pl.BlockSpec((bn, FDIM), lambda i: (i, 0)),
        out_shape=jax.ShapeDtypeStruct((N_NODES, FDIM), jnp.float32),
    )(partials, weight)


def kernel(features, adj_indices, adj_values, weight):
    idx = adj_indices.astype(jnp.int32)
    n_edges = idx.shape[1]
    gran = NW * GRP * CHUNK
    n_pad = (n_edges + gran - 1) // gran * gran
    pad = n_pad - n_edges
    n_chunks = n_pad // CHUNK
    spread = (jnp.arange(pad, dtype=jnp.int32) * 8) % N_NODES
    row = jnp.concatenate([idx[0], spread]).reshape(n_chunks, CHUNK)
    col = jnp.concatenate([idx[1], spread]).reshape(n_chunks, CHUNK)
    val = jnp.pad(adj_values, (0, pad)).reshape(n_chunks, CHUNK)
    partials = _sc_aggregate(row, col, val, features)
    return _tc_combine_matmul(partials, weight)


# speedup vs baseline: 2.9826x; 1.1990x over previous
import functools

import jax
import jax.numpy as jnp
from jax import lax
from jax.experimental import pallas as pl
from jax.experimental.pallas import tpu as pltpu
from jax.experimental.pallas import tpu_sc as plsc

N_NODES = 10000
FDIM = 128
CHUNK = 128
NC = 2
NS = 16
NW = NC * NS
ROWS_MAIN = (N_NODES // NS) // 8 * 8
ROWS_TAIL = N_NODES - NS * ROWS_MAIN
GRP = 8
SUP = 1


def _sc_aggregate(row2d, col2d, val2d, features):
    cpw = row2d.shape[0] // NW

    mesh = plsc.VectorSubcoreMesh(core_axis_name="c", subcore_axis_name="s")

    @functools.partial(
        pl.kernel,
        mesh=mesh,
        out_type=jax.ShapeDtypeStruct((NC, N_NODES, FDIM), jnp.float32),
        scratch_types=[
            pltpu.VMEM_SHARED((N_NODES, FDIM), jnp.float32),
            pltpu.VMEM((GRP, CHUNK), jnp.int32),
            pltpu.VMEM((GRP, CHUNK), jnp.int32),
            pltpu.VMEM((GRP, CHUNK), jnp.float32),
            pltpu.VMEM((2, CHUNK, FDIM), jnp.float32),
            pltpu.SemaphoreType.DMA,
            pltpu.SemaphoreType.DMA,
            pltpu.SemaphoreType.DMA,
            pltpu.SemaphoreType.DMA,
        ],
    )
    def agg(row_hbm, col_hbm, val_hbm, feat_hbm, out_hbm, acc, colv, rowv,
            valv, grows, gsem0, gsem1, ssem0, ssem1):
        cc = lax.axis_index("c")
        sid = lax.axis_index("s")
        wid = sid * NC + cc
        chunk_base = wid * cpw

        def zrow(r, _):
            def zcol(i, _):
                grows[0, r, pl.ds(i * 16, 16)] = jnp.zeros((16,), jnp.float32)
                return 0
            return lax.fori_loop(0, FDIM // 16, zcol, 0)
        lax.fori_loop(0, CHUNK, zrow, 0)

        base_row = sid * ROWS_MAIN
        for j in range(ROWS_MAIN // CHUNK):
            pltpu.sync_copy(grows.at[0],
                            acc.at[pl.ds(base_row + j * CHUNK, CHUNK)])
        tail = ROWS_MAIN - (ROWS_MAIN // CHUNK) * CHUNK
        pltpu.sync_copy(
            grows.at[0, pl.ds(0, tail)],
            acc.at[pl.ds(base_row + (ROWS_MAIN // CHUNK) * CHUNK, tail)])

        @pl.when(sid == 0)
        def _():
            pltpu.sync_copy(grows.at[0, pl.ds(0, ROWS_TAIL)],
                            acc.at[pl.ds(NS * ROWS_MAIN, ROWS_TAIL)])
        plsc.subcore_barrier()

        def group_body(g, _):
            gbase = chunk_base + g * GRP
            pltpu.sync_copy(col_hbm.at[pl.ds(gbase, GRP)], colv)
            pltpu.sync_copy(row_hbm.at[pl.ds(gbase, GRP)], rowv)
            pltpu.sync_copy(val_hbm.at[pl.ds(gbase, GRP)], valv)
            def scale(s, b):
                def grp_body(grp, _):
                    vv = valv[s, pl.ds(grp * 16, 16)]
                    for lane in range(16):
                        v = vv[lane]
                        e = grp * 16 + lane
                        for kk in range(FDIM // 16):
                            grows[b, e, pl.ds(kk * 16, 16)] = (
                                grows[b, e, pl.ds(kk * 16, 16)] * v)
                    return 0
                lax.fori_loop(0, CHUNK // 16, grp_body, 0)

            def pair_body(p, _):
                s0 = p * 2
                s1 = s0 + 1
                g0 = pltpu.async_copy(feat_hbm.at[colv.at[s0]], grows.at[0],
                                      gsem0)
                g1 = pltpu.async_copy(feat_hbm.at[colv.at[s1]], grows.at[1],
                                      gsem1)
                g0.wait()
                scale(s0, 0)
                c0 = pltpu.async_copy(grows.at[0], acc.at[rowv.at[s0]], ssem0,
                                      add=True)
                g1.wait()
                scale(s1, 1)
                c1 = pltpu.async_copy(grows.at[1], acc.at[rowv.at[s1]], ssem1,
                                      add=True)
                c0.wait()
                c1.wait()
                return 0

            lax.fori_loop(0, GRP // 2, pair_body, 0)
            return 0

        lax.fori_loop(0, cpw // GRP, group_body, 0)
        plsc.subcore_barrier()

        pltpu.sync_copy(acc.at[pl.ds(base_row, ROWS_MAIN)],
                        out_hbm.at[cc, pl.ds(base_row, ROWS_MAIN)])

        @pl.when(sid == 0)
        def _():
            pltpu.sync_copy(acc.at[pl.ds(NS * ROWS_MAIN, ROWS_TAIL)],
                            out_hbm.at[cc, pl.ds(NS * ROWS_MAIN, ROWS_TAIL)])

    return agg(row2d, col2d, val2d, features)


def _tc_combine_matmul(partials, weight):
    bn = 1000

    def body(p_ref, w_ref, o_ref):
        s = p_ref[0] + p_ref[1]
        o_ref[...] = jnp.maximum(
            jnp.dot(s, w_ref[...], preferred_element_type=jnp.float32), 0.0)

    return pl.pallas_call(
        body,
        grid=(N_NODES // bn,),
        in_specs=[
            pl.BlockSpec((NC, bn, FDIM), lambda i: (0, i, 0)),
            pl.BlockSpec((FDIM, FDIM), lambda i: (0, 0)),
        ],
        out_specs=pl.BlockSpec((bn, FDIM), lambda i: (i, 0)),
        out_shape=jax.ShapeDtypeStruct((N_NODES, FDIM), jnp.float32),
    )(partials, weight)


def kernel(features, adj_indices, adj_values, weight):
    idx = adj_indices.astype(jnp.int32)
    n_edges = idx.shape[1]
    gran = NW * GRP * CHUNK
    n_pad = (n_edges + gran - 1) // gran * gran
    pad = n_pad - n_edges
    n_chunks = n_pad // CHUNK
    spread = (jnp.arange(pad, dtype=jnp.int32) * 8) % N_NODES
    row = jnp.concatenate([idx[0], spread]).reshape(n_chunks, CHUNK)
    col = jnp.concatenate([idx[1], spread]).reshape(n_chunks, CHUNK)
    val = jnp.pad(adj_values, (0, pad)).reshape(n_chunks, CHUNK)
    partials = _sc_aggregate(row, col, val, features)
    return _tc_combine_matmul(partials, weight)


# speedup vs baseline: 3.0244x; 1.0140x over previous
import functools

import jax
import jax.numpy as jnp
from jax import lax
from jax.experimental import pallas as pl
from jax.experimental.pallas import tpu as pltpu
from jax.experimental.pallas import tpu_sc as plsc

N_NODES = 10000
FDIM = 128
CHUNK = 128
NC = 2
NS = 16
NW = NC * NS
ROWS_MAIN = (N_NODES // NS) // 8 * 8
ROWS_TAIL = N_NODES - NS * ROWS_MAIN
GRP = 8
SUP = 1


def _sc_aggregate(row2d, col2d, val2d, features):
    cpw = row2d.shape[0] // NW

    mesh = plsc.VectorSubcoreMesh(core_axis_name="c", subcore_axis_name="s")

    @functools.partial(
        pl.kernel,
        mesh=mesh,
        out_type=jax.ShapeDtypeStruct((NC, N_NODES, FDIM), jnp.float32),
        scratch_types=[
            pltpu.VMEM_SHARED((N_NODES, FDIM), jnp.float32),
            pltpu.VMEM((GRP, CHUNK), jnp.int32),
            pltpu.VMEM((GRP, CHUNK), jnp.int32),
            pltpu.VMEM((GRP, CHUNK), jnp.float32),
            pltpu.VMEM((2, CHUNK, FDIM), jnp.float32),
            pltpu.SemaphoreType.DMA,
            pltpu.SemaphoreType.DMA,
            pltpu.SemaphoreType.DMA,
            pltpu.SemaphoreType.DMA,
        ],
    )
    def agg(row_hbm, col_hbm, val_hbm, feat_hbm, out_hbm, acc, colv, rowv,
            valv, grows, gsem0, gsem1, ssem0, ssem1):
        cc = lax.axis_index("c")
        sid = lax.axis_index("s")
        wid = sid * NC + cc
        chunk_base = wid * cpw

        def zrow(r, _):
            def zcol(i, _):
                grows[0, r, pl.ds(i * 16, 16)] = jnp.zeros((16,), jnp.float32)
                return 0
            return lax.fori_loop(0, FDIM // 16, zcol, 0)
        lax.fori_loop(0, CHUNK, zrow, 0)

        base_row = sid * ROWS_MAIN
        for j in range(ROWS_MAIN // CHUNK):
            pltpu.sync_copy(grows.at[0],
                            acc.at[pl.ds(base_row + j * CHUNK, CHUNK)])
        tail = ROWS_MAIN - (ROWS_MAIN // CHUNK) * CHUNK
        pltpu.sync_copy(
            grows.at[0, pl.ds(0, tail)],
            acc.at[pl.ds(base_row + (ROWS_MAIN // CHUNK) * CHUNK, tail)])

        @pl.when(sid == 0)
        def _():
            pltpu.sync_copy(grows.at[0, pl.ds(0, ROWS_TAIL)],
                            acc.at[pl.ds(NS * ROWS_MAIN, ROWS_TAIL)])
        plsc.subcore_barrier()

        def group_body(g, _):
            gbase = chunk_base + g * GRP
            pltpu.sync_copy(col_hbm.at[pl.ds(gbase, GRP)], colv)
            pltpu.sync_copy(row_hbm.at[pl.ds(gbase, GRP)], rowv)
            pltpu.sync_copy(val_hbm.at[pl.ds(gbase, GRP)], valv)
            def scale(s, b):
                def grp_body(grp, _):
                    vv = valv[s, pl.ds(grp * 16, 16)]
                    for lane in range(16):
                        v = vv[lane]
                        e = grp * 16 + lane
                        for kk in range(FDIM // 16):
                            grows[b, e, pl.ds(kk * 16, 16)] = (
                                grows[b, e, pl.ds(kk * 16, 16)] * v)
                    return 0
                lax.fori_loop(0, CHUNK // 16, grp_body, 0)

            def pair_body(p, _):
                s0 = p * 2
                s1 = s0 + 1

                @pl.when((p >= 1) | (g >= 1))
                def _():
                    pltpu.make_async_copy(grows.at[1], acc.at[rowv.at[s0]],
                                          ssem1).wait()
                g0 = pltpu.async_copy(feat_hbm.at[colv.at[s0]], grows.at[0],
                                      gsem0)
                g1 = pltpu.async_copy(feat_hbm.at[colv.at[s1]], grows.at[1],
                                      gsem1)
                g0.wait()
                scale(s0, 0)
                c0 = pltpu.async_copy(grows.at[0], acc.at[rowv.at[s0]], ssem0,
                                      add=True)
                g1.wait()
                scale(s1, 1)
                pltpu.async_copy(grows.at[1], acc.at[rowv.at[s1]], ssem1,
                                 add=True)
                c0.wait()
                return 0

            lax.fori_loop(0, GRP // 2, pair_body, 0)
            return 0

        lax.fori_loop(0, cpw // GRP, group_body, 0)
        pltpu.make_async_copy(grows.at[1], acc.at[rowv.at[1]], ssem1).wait()
        plsc.subcore_barrier()

        pltpu.sync_copy(acc.at[pl.ds(base_row, ROWS_MAIN)],
                        out_hbm.at[cc, pl.ds(base_row, ROWS_MAIN)])

        @pl.when(sid == 0)
        def _():
            pltpu.sync_copy(acc.at[pl.ds(NS * ROWS_MAIN, ROWS_TAIL)],
                            out_hbm.at[cc, pl.ds(NS * ROWS_MAIN, ROWS_TAIL)])

    return agg(row2d, col2d, val2d, features)


def _tc_combine_matmul(partials, weight):
    bn = 1000

    def body(p_ref, w_ref, o_ref):
        s = p_ref[0] + p_ref[1]
        o_ref[...] = jnp.maximum(
            jnp.dot(s, w_ref[...], preferred_element_type=jnp.float32), 0.0)

    return pl.pallas_call(
        body,
        grid=(N_NODES // bn,),
        in_specs=[
            pl.BlockSpec((NC, bn, FDIM), lambda i: (0, i, 0)),
            pl.BlockSpec((FDIM, FDIM), lambda i: (0, 0)),
        ],
        out_specs=pl.BlockSpec((bn, FDIM), lambda i: (i, 0)),
        out_shape=jax.ShapeDtypeStruct((N_NODES, FDIM), jnp.float32),
    )(partials, weight)


def kernel(features, adj_indices, adj_values, weight):
    idx = adj_indices.astype(jnp.int32)
    n_edges = idx.shape[1]
    gran = NW * GRP * CHUNK
    n_pad = (n_edges + gran - 1) // gran * gran
    pad = n_pad - n_edges
    n_chunks = n_pad // CHUNK
    spread = (jnp.arange(pad, dtype=jnp.int32) * 8) % N_NODES
    row = jnp.concatenate([idx[0], spread]).reshape(n_chunks, CHUNK)
    col = jnp.concatenate([idx[1], spread]).reshape(n_chunks, CHUNK)
    val = jnp.pad(adj_values, (0, pad)).reshape(n_chunks, CHUNK)
    partials = _sc_aggregate(row, col, val, features)
    return _tc_combine_matmul(partials, weight)
